# baseline (speedup 1.0000x reference)
import jax
import jax.numpy as jnp
from jax.experimental import pallas as pl
from jax.experimental.pallas import tpu as pltpu

B, N, D = 64, 128, 128
F32 = jnp.float32


def _gnn_kernel(sizes1_ref, sizes2_ref,
                f1_ref, a1_ref, f2_ref, a2_ref,
                w1_ref, b1_ref, w2_ref, b2_ref, wa_ref, ba_ref,
                wc1_ref, wc2_ref, bc_ref,
                out_ref):
    b = pl.program_id(0)
    size1 = sizes1_ref[b]
    size2 = sizes2_ref[b]

    lane_iota = jax.lax.broadcasted_iota(jnp.int32, (1, N), 1)
    cm1 = (lane_iota < size1).astype(F32)
    cm2 = (lane_iota < size2).astype(F32)

    A1 = a1_ref[0] * cm1
    A2 = a2_ref[0] * cm2

    x12 = jnp.concatenate([f1_ref[0], f2_ref[0]], axis=0)

    dot = lambda a, b_: jnp.dot(a, b_, preferred_element_type=F32)

    h = dot(x12, w1_ref[...]) + b1_ref[...]
    t1 = jnp.maximum(dot(A1, h[:N]), 0.0)
    t2 = jnp.maximum(dot(A2, h[N:]), 0.0)

    u = dot(jnp.concatenate([t1, t2], axis=0), w2_ref[...]) + b2_ref[...]
    v1 = jnp.maximum(dot(A1, u[:N]), 0.0)
    v2 = jnp.maximum(dot(A2, u[N:]), 0.0)

    s1 = dot(cm1, v1)
    s2 = dot(cm2, v2)

    d1 = 1.0 / jnp.maximum(size1, 1).astype(F32)
    d2 = 1.0 / jnp.maximum(size2, 1).astype(F32)
    g1 = (size1 > 0).astype(F32)
    g2 = (size2 > 0).astype(F32)

    emb1 = dot(s1, wa_ref[...]) * d1 + ba_ref[...] * g1
    emb2 = dot(s2, wa_ref[...]) * d2 + ba_ref[...] * g2

    r = dot(emb1, wc1_ref[...]) + dot(emb2, wc2_ref[...]) + bc_ref[...]
    out_ref[0, 0, :] = r[0]


def kernel(feats_1, adjs_1, feats_2, adjs_2, sizes_1, sizes_2,
           W1, b1, W2, b2, Wa, ba, Wc, bc):
    sizes_1 = sizes_1.astype(jnp.int32)
    sizes_2 = sizes_2.astype(jnp.int32)

    C = Wc.shape[1]
    wc1 = jnp.pad(Wc[:D], ((0, 0), (0, D - C)))
    wc2 = jnp.pad(Wc[D:], ((0, 0), (0, D - C)))
    bcp = jnp.pad(bc, (0, D - C)).reshape(1, D)

    batch_spec = pl.BlockSpec((1, N, D), lambda b: (b, 0, 0))
    w_spec = pl.BlockSpec((D, D), lambda b: (0, 0))
    row_spec = pl.BlockSpec((1, D), lambda b: (0, 0))
    smem_spec = pl.BlockSpec(memory_space=pltpu.SMEM)

    out3 = pl.pallas_call(
        _gnn_kernel,
        grid=(B,),
        in_specs=[smem_spec, smem_spec,
                  batch_spec, batch_spec, batch_spec, batch_spec,
                  w_spec, row_spec, w_spec, row_spec, w_spec, row_spec,
                  w_spec, w_spec, row_spec],
        out_specs=pl.BlockSpec((1, 1, D), lambda b: (b, 0, 0)),
        out_shape=jax.ShapeDtypeStruct((B, 1, D), F32),
        compiler_params=pltpu.CompilerParams(
            dimension_semantics=("parallel",)),
    )(sizes_1, sizes_2,
      feats_1, adjs_1, feats_2, adjs_2,
      W1, b1.reshape(1, D), W2, b2.reshape(1, D), Wa, ba.reshape(1, D),
      wc1, wc2, bcp)

    return out3.reshape(B, D)[:, :C]


# speedup vs baseline: 3.6672x; 3.6672x over previous
import jax
import jax.numpy as jnp
from jax.experimental import pallas as pl
from jax.experimental.pallas import tpu as pltpu

B, N, D = 64, 128, 128
BB = 8
F32 = jnp.float32


def _gnn_kernel(sizes1_ref, sizes2_ref,
                f1_ref, a1_ref, f2_ref, a2_ref,
                w1_ref, b1_ref, w2_ref, b2_ref, wa_ref, ba_ref,
                wc1_ref, wc2_ref, bc_ref,
                out_ref):
    pid = pl.program_id(0)
    lane_iota = jax.lax.broadcasted_iota(jnp.int32, (1, N), 1)
    dot = lambda a, b_: jnp.dot(a, b_, preferred_element_type=F32)

    sizes1 = [sizes1_ref[pid * BB + i] for i in range(BB)]
    sizes2 = [sizes2_ref[pid * BB + i] for i in range(BB)]
    cms = ([(lane_iota < s).astype(F32) for s in sizes1]
           + [(lane_iota < s).astype(F32) for s in sizes2])

    As = ([a1_ref[i] * cms[i] for i in range(BB)]
          + [a2_ref[i] * cms[BB + i] for i in range(BB)])

    x = jnp.concatenate([f1_ref[...].reshape(BB * N, D),
                         f2_ref[...].reshape(BB * N, D)], axis=0)

    h = dot(x, w1_ref[...]) + b1_ref[...]
    t = [jnp.maximum(dot(As[k], h[k * N:(k + 1) * N]), 0.0)
         for k in range(2 * BB)]

    u = dot(jnp.concatenate(t, axis=0), w2_ref[...]) + b2_ref[...]
    v = [jnp.maximum(dot(As[k], u[k * N:(k + 1) * N]), 0.0)
         for k in range(2 * BB)]

    S1 = jnp.concatenate([dot(cms[k], v[k]) for k in range(BB)], axis=0)
    S2 = jnp.concatenate([dot(cms[BB + k], v[BB + k]) for k in range(BB)],
                         axis=0)

    inv1 = jnp.concatenate(
        [(1.0 / jnp.maximum(s, 1).astype(F32)).reshape(1, 1) for s in sizes1],
        axis=0)
    inv2 = jnp.concatenate(
        [(1.0 / jnp.maximum(s, 1).astype(F32)).reshape(1, 1) for s in sizes2],
        axis=0)
    g1 = jnp.concatenate(
        [(s > 0).astype(F32).reshape(1, 1) for s in sizes1], axis=0)
    g2 = jnp.concatenate(
        [(s > 0).astype(F32).reshape(1, 1) for s in sizes2], axis=0)

    emb1 = dot(S1, wa_ref[...]) * inv1 + ba_ref[...] * g1
    emb2 = dot(S2, wa_ref[...]) * inv2 + ba_ref[...] * g2

    r = dot(emb1, wc1_ref[...]) + dot(emb2, wc2_ref[...]) + bc_ref[...]
    out_ref[...] = r.reshape(BB, 1, D)


def kernel(feats_1, adjs_1, feats_2, adjs_2, sizes_1, sizes_2,
           W1, b1, W2, b2, Wa, ba, Wc, bc):
    sizes_1 = sizes_1.astype(jnp.int32)
    sizes_2 = sizes_2.astype(jnp.int32)

    C = Wc.shape[1]
    wc1 = jnp.pad(Wc[:D], ((0, 0), (0, D - C)))
    wc2 = jnp.pad(Wc[D:], ((0, 0), (0, D - C)))
    bcp = jnp.pad(bc, (0, D - C)).reshape(1, D)

    batch_spec = pl.BlockSpec((BB, N, D), lambda b: (b, 0, 0))
    w_spec = pl.BlockSpec((D, D), lambda b: (0, 0))
    row_spec = pl.BlockSpec((1, D), lambda b: (0, 0))
    smem_spec = pl.BlockSpec(memory_space=pltpu.SMEM)

    out3 = pl.pallas_call(
        _gnn_kernel,
        grid=(B // BB,),
        in_specs=[smem_spec, smem_spec,
                  batch_spec, batch_spec, batch_spec, batch_spec,
                  w_spec, row_spec, w_spec, row_spec, w_spec, row_spec,
                  w_spec, w_spec, row_spec],
        out_specs=pl.BlockSpec((BB, 1, D), lambda b: (b, 0, 0)),
        out_shape=jax.ShapeDtypeStruct((B, 1, D), F32),
        compiler_params=pltpu.CompilerParams(
            dimension_semantics=("parallel",)),
    )(sizes_1, sizes_2,
      feats_1, adjs_1, feats_2, adjs_2,
      W1, b1.reshape(1, D), W2, b2.reshape(1, D), Wa, ba.reshape(1, D),
      wc1, wc2, bcp)

    return out3.reshape(B, D)[:, :C]


# speedup vs baseline: 4.1150x; 1.1221x over previous
import jax
import jax.numpy as jnp
from jax.experimental import pallas as pl
from jax.experimental.pallas import tpu as pltpu

B, N, D = 64, 128, 128
BB = 16
F32 = jnp.float32


def _gnn_kernel(sizes1_ref, sizes2_ref,
                f1_ref, a1_ref, f2_ref, a2_ref,
                w1_ref, b1_ref, w2_ref, b2_ref, wa_ref, ba_ref,
                wc1_ref, wc2_ref, bc_ref,
                out_ref):
    pid = pl.program_id(0)
    lane_iota = jax.lax.broadcasted_iota(jnp.int32, (1, N), 1)
    dot = lambda a, b_: jnp.dot(a, b_, preferred_element_type=F32)

    sizes1 = [sizes1_ref[pid * BB + i] for i in range(BB)]
    sizes2 = [sizes2_ref[pid * BB + i] for i in range(BB)]
    cms = ([(lane_iota < s).astype(F32) for s in sizes1]
           + [(lane_iota < s).astype(F32) for s in sizes2])

    As = ([a1_ref[i] * cms[i] for i in range(BB)]
          + [a2_ref[i] * cms[BB + i] for i in range(BB)])

    x = jnp.concatenate([f1_ref[...].reshape(BB * N, D),
                         f2_ref[...].reshape(BB * N, D)], axis=0)

    h = dot(x, w1_ref[...]) + b1_ref[...]
    t = [jnp.maximum(dot(As[k], h[k * N:(k + 1) * N]), 0.0)
         for k in range(2 * BB)]

    u = dot(jnp.concatenate(t, axis=0), w2_ref[...]) + b2_ref[...]
    v = [jnp.maximum(dot(As[k], u[k * N:(k + 1) * N]), 0.0)
         for k in range(2 * BB)]

    S1 = jnp.concatenate([dot(cms[k], v[k]) for k in range(BB)], axis=0)
    S2 = jnp.concatenate([dot(cms[BB + k], v[BB + k]) for k in range(BB)],
                         axis=0)

    inv1 = jnp.concatenate(
        [(1.0 / jnp.maximum(s, 1).astype(F32)).reshape(1, 1) for s in sizes1],
        axis=0)
    inv2 = jnp.concatenate(
        [(1.0 / jnp.maximum(s, 1).astype(F32)).reshape(1, 1) for s in sizes2],
        axis=0)
    g1 = jnp.concatenate(
        [(s > 0).astype(F32).reshape(1, 1) for s in sizes1], axis=0)
    g2 = jnp.concatenate(
        [(s > 0).astype(F32).reshape(1, 1) for s in sizes2], axis=0)

    emb1 = dot(S1, wa_ref[...]) * inv1 + ba_ref[...] * g1
    emb2 = dot(S2, wa_ref[...]) * inv2 + ba_ref[...] * g2

    r = dot(emb1, wc1_ref[...]) + dot(emb2, wc2_ref[...]) + bc_ref[...]
    out_ref[...] = r.reshape(BB, 1, D)


def kernel(feats_1, adjs_1, feats_2, adjs_2, sizes_1, sizes_2,
           W1, b1, W2, b2, Wa, ba, Wc, bc):
    sizes_1 = sizes_1.astype(jnp.int32)
    sizes_2 = sizes_2.astype(jnp.int32)

    C = Wc.shape[1]
    wc1 = jnp.pad(Wc[:D], ((0, 0), (0, D - C)))
    wc2 = jnp.pad(Wc[D:], ((0, 0), (0, D - C)))
    bcp = jnp.pad(bc, (0, D - C)).reshape(1, D)

    batch_spec = pl.BlockSpec((BB, N, D), lambda b: (b, 0, 0))
    w_spec = pl.BlockSpec((D, D), lambda b: (0, 0))
    row_spec = pl.BlockSpec((1, D), lambda b: (0, 0))
    smem_spec = pl.BlockSpec(memory_space=pltpu.SMEM)

    out3 = pl.pallas_call(
        _gnn_kernel,
        grid=(B // BB,),
        in_specs=[smem_spec, smem_spec,
                  batch_spec, batch_spec, batch_spec, batch_spec,
                  w_spec, row_spec, w_spec, row_spec, w_spec, row_spec,
                  w_spec, w_spec, row_spec],
        out_specs=pl.BlockSpec((BB, 1, D), lambda b: (b, 0, 0)),
        out_shape=jax.ShapeDtypeStruct((B, 1, D), F32),
        compiler_params=pltpu.CompilerParams(
            dimension_semantics=("parallel",)),
    )(sizes_1, sizes_2,
      feats_1, adjs_1, feats_2, adjs_2,
      W1, b1.reshape(1, D), W2, b2.reshape(1, D), Wa, ba.reshape(1, D),
      wc1, wc2, bcp)

    return out3.reshape(B, D)[:, :C]
